# baseline (speedup 1.0000x reference)
import functools

import jax
import jax.numpy as jnp
from jax import lax
from jax.experimental import pallas as pl
from jax.experimental.pallas import tpu as pltpu
from jax.experimental.pallas import tpu_sc as plsc

_B = 16384
_DIM = 64
_MARGIN = 1.0
_NC = 2
_NS = 16
_NW = _NC * _NS
_PER_W = _B // _NW
_CHUNK = 128
_NCHUNK = _PER_W // _CHUNK


def _rsqrt16(x):
    i = lax.bitcast_convert_type(x, jnp.int32)
    i = jnp.int32(0x5F3759DF) - (i >> 1)
    y = lax.bitcast_convert_type(i, jnp.float32)
    xh = 0.5 * x
    for _ in range(3):
        y = y * (1.5 - xh * y * y)
    return y


def _block_score(buf_h, buf_t, buf_r, buf_u, rows):
    zeros = jnp.zeros((16,), jnp.float32)

    def dots_body(d, carry):
        nn, hh, tt, rr, hn, tn = carry
        col = jnp.full((16,), d, jnp.int32)
        hv = plsc.load_gather(buf_h, [rows, col])
        tv = plsc.load_gather(buf_t, [rows, col])
        rv = plsc.load_gather(buf_r, [rows, col])
        uv = plsc.load_gather(buf_u, [rows, col])
        return (nn + uv * uv, hh + hv * hv, tt + tv * tv,
                rr + rv * rv, hn + hv * uv, tn + tv * uv)

    nn, hh, tt, rr, hn, tn = lax.fori_loop(
        0, _DIM, dots_body, (zeros,) * 6, unroll=4)

    eps = jnp.float32(1e-12)
    inv_n = _rsqrt16(jnp.maximum(nn, eps))
    s = nn * inv_n * inv_n
    a_h = hn * inv_n
    a_t = tn * inv_n
    php = hh - a_h * a_h * (2.0 - s)
    ptp = tt - a_t * a_t * (2.0 - s)
    inv_h = _rsqrt16(jnp.maximum(php, eps))
    inv_t = _rsqrt16(jnp.maximum(ptp, eps))
    inv_r = _rsqrt16(jnp.maximum(rr, eps))
    gamma = inv_h * (a_h * inv_n) - inv_t * (a_t * inv_n)

    def score_body(d, acc):
        col = jnp.full((16,), d, jnp.int32)
        hv = plsc.load_gather(buf_h, [rows, col])
        tv = plsc.load_gather(buf_t, [rows, col])
        rv = plsc.load_gather(buf_r, [rows, col])
        uv = plsc.load_gather(buf_u, [rows, col])
        c = inv_h * hv - inv_t * tv + inv_r * rv - gamma * uv
        return acc + jnp.abs(c)

    return lax.fori_loop(0, _DIM, score_body, zeros, unroll=4)


def _transh_body(ph_hbm, pt_hbm, pr_hbm, nh_hbm, nt_hbm, nr_hbm,
                 ent_hbm, rel_hbm, nrm_hbm, out_hbm,
                 iph, ipt, ipr, inh, int_, inr,
                 bph, bpt, bpr, bpu, bnh, bnt, bnr, bnu,
                 out_v, sem):
    wid = lax.axis_index("s") * _NC + lax.axis_index("c")
    base = wid * _PER_W
    iota = lax.iota(jnp.int32, 16)

    acc = jnp.zeros((16,), jnp.float32)
    for ch in range(_NCHUNK):
        off = base + ch * _CHUNK
        pltpu.sync_copy(ph_hbm.at[pl.ds(off, _CHUNK)], iph)
        pltpu.sync_copy(pt_hbm.at[pl.ds(off, _CHUNK)], ipt)
        pltpu.sync_copy(pr_hbm.at[pl.ds(off, _CHUNK)], ipr)
        pltpu.sync_copy(nh_hbm.at[pl.ds(off, _CHUNK)], inh)
        pltpu.sync_copy(nt_hbm.at[pl.ds(off, _CHUNK)], int_)
        pltpu.sync_copy(nr_hbm.at[pl.ds(off, _CHUNK)], inr)
        copies = [
            pltpu.async_copy(ent_hbm.at[iph], bph, sem),
            pltpu.async_copy(ent_hbm.at[ipt], bpt, sem),
            pltpu.async_copy(rel_hbm.at[ipr], bpr, sem),
            pltpu.async_copy(nrm_hbm.at[ipr], bpu, sem),
            pltpu.async_copy(ent_hbm.at[inh], bnh, sem),
            pltpu.async_copy(ent_hbm.at[int_], bnt, sem),
            pltpu.async_copy(rel_hbm.at[inr], bnr, sem),
            pltpu.async_copy(nrm_hbm.at[inr], bnu, sem),
        ]
        for c in copies:
            c.wait()

        def block_body(b, a):
            rows = b * 16 + iota
            sp = _block_score(bph, bpt, bpr, bpu, rows)
            sn = _block_score(bnh, bnt, bnr, bnu, rows)
            return a + jnp.maximum(sp - sn + _MARGIN, 0.0)

        acc = lax.fori_loop(0, _CHUNK // 16, block_body, acc)

    out_v[...] = acc
    pltpu.sync_copy(out_v, out_hbm.at[wid])


@functools.partial(jax.jit, static_argnames=())
def _transh_sc(pos_h, pos_t, pos_r, neg_h, neg_t, neg_r,
               ent_emb, rel_emb, norm_vec):
    f32 = jnp.float32
    i32 = jnp.int32
    call = pl.kernel(
        _transh_body,
        out_type=jax.ShapeDtypeStruct((_NW, 16), f32),
        mesh=plsc.VectorSubcoreMesh(core_axis_name="c", subcore_axis_name="s"),
        compiler_params=pltpu.CompilerParams(
            needs_layout_passes=False, use_tc_tiling_on_sc=False),
        scratch_types=[
            pltpu.VMEM((_CHUNK,), i32),
            pltpu.VMEM((_CHUNK,), i32),
            pltpu.VMEM((_CHUNK,), i32),
            pltpu.VMEM((_CHUNK,), i32),
            pltpu.VMEM((_CHUNK,), i32),
            pltpu.VMEM((_CHUNK,), i32),
            pltpu.VMEM((_CHUNK, _DIM), f32),
            pltpu.VMEM((_CHUNK, _DIM), f32),
            pltpu.VMEM((_CHUNK, _DIM), f32),
            pltpu.VMEM((_CHUNK, _DIM), f32),
            pltpu.VMEM((_CHUNK, _DIM), f32),
            pltpu.VMEM((_CHUNK, _DIM), f32),
            pltpu.VMEM((_CHUNK, _DIM), f32),
            pltpu.VMEM((_CHUNK, _DIM), f32),
            pltpu.VMEM((16,), f32),
            pltpu.SemaphoreType.DMA,
        ],
    )
    return call(pos_h, pos_t, pos_r, neg_h, neg_t, neg_r,
                ent_emb, rel_emb, norm_vec)


def kernel(pos_h, pos_t, pos_r, neg_h, neg_t, neg_r,
           ent_emb, rel_emb, norm_vec):
    partials = _transh_sc(pos_h, pos_t, pos_r, neg_h, neg_t, neg_r,
                          ent_emb, rel_emb, norm_vec)
    return jnp.sum(partials) / _B


# speedup vs baseline: 2.0410x; 2.0410x over previous
import functools

import jax
import jax.numpy as jnp
from jax import lax
from jax.experimental import pallas as pl
from jax.experimental.pallas import tpu as pltpu
from jax.experimental.pallas import tpu_sc as plsc

_B = 16384
_DIM = 64
_MARGIN = 1.0
_NC = 2
_NS = 16
_NW = _NC * _NS
_PER_W = _B // _NW
_C = 64
_NCHUNK = _PER_W // _C
_EPS = 1e-12


def _rsqrt16(x):
    i = lax.bitcast_convert_type(x, jnp.int32)
    i = jnp.int32(0x5F3759DF) - (i >> 1)
    y = lax.bitcast_convert_type(i, jnp.float32)
    xh = 0.5 * x
    for _ in range(3):
        y = y * (1.5 - xh * y * y)
    return y


def _dot4(a, b):
    return jnp.sum(a[0] * b[0] + a[1] * b[1] + a[2] * b[2] + a[3] * b[3])


def _load4(ref, r):
    return [ref[r, pl.ds(16 * j, 16)] for j in range(4)]


def _transh_body(ph_hbm, pt_hbm, nh_hbm, nt_hbm, pr_hbm, nr_hbm,
                 ent_hbm, rn_hbm, out_hbm,
                 iph, ipt, inh, int_, ipr, inr,
                 bph, bpt, bnh, bnt, bpr, bnr,
                 dots, coef, sps, out_v, sem):
    wid = lax.axis_index("s") * _NC + lax.axis_index("c")
    base = wid * _PER_W
    iota = lax.iota(jnp.int32, 16)
    zeros = jnp.zeros((16,), jnp.float32)

    def chunk_body(ch, acc):
        off = base + ch * _C
        pltpu.sync_copy(ph_hbm.at[pl.ds(off, _C)], iph)
        pltpu.sync_copy(pt_hbm.at[pl.ds(off, _C)], ipt)
        pltpu.sync_copy(nh_hbm.at[pl.ds(off, _C)], inh)
        pltpu.sync_copy(nt_hbm.at[pl.ds(off, _C)], int_)
        pltpu.sync_copy(pr_hbm.at[pl.ds(off, _C)], ipr)
        pltpu.sync_copy(nr_hbm.at[pl.ds(off, _C)], inr)

        def issue(g, carry):
            vph = iph[pl.ds(g * 16, 16)]
            vpt = ipt[pl.ds(g * 16, 16)]
            vnh = inh[pl.ds(g * 16, 16)]
            vnt = int_[pl.ds(g * 16, 16)]
            vpr = ipr[pl.ds(g * 16, 16)]
            vnr = inr[pl.ds(g * 16, 16)]
            for k in range(16):
                r = g * 16 + k
                pltpu.async_copy(ent_hbm.at[pl.ds(vph[k], 1), :],
                                 bph.at[pl.ds(r, 1), :], sem)
                pltpu.async_copy(ent_hbm.at[pl.ds(vpt[k], 1), :],
                                 bpt.at[pl.ds(r, 1), :], sem)
                pltpu.async_copy(ent_hbm.at[pl.ds(vnh[k], 1), :],
                                 bnh.at[pl.ds(r, 1), :], sem)
                pltpu.async_copy(ent_hbm.at[pl.ds(vnt[k], 1), :],
                                 bnt.at[pl.ds(r, 1), :], sem)
                pltpu.async_copy(rn_hbm.at[pl.ds(2 * vpr[k], 2), :],
                                 bpr.at[pl.ds(2 * r, 2), :], sem)
                pltpu.async_copy(rn_hbm.at[pl.ds(2 * vnr[k], 2), :],
                                 bnr.at[pl.ds(2 * r, 2), :], sem)
            return carry

        lax.fori_loop(0, _C // 16, issue, 0)
        pltpu.make_async_copy(ent_hbm.at[pl.ds(0, _C), :], bph, sem).wait()
        pltpu.make_async_copy(ent_hbm.at[pl.ds(0, _C), :], bpt, sem).wait()
        pltpu.make_async_copy(ent_hbm.at[pl.ds(0, _C), :], bnh, sem).wait()
        pltpu.make_async_copy(ent_hbm.at[pl.ds(0, _C), :], bnt, sem).wait()
        pltpu.make_async_copy(rn_hbm.at[pl.ds(0, 2 * _C), :], bpr, sem).wait()
        pltpu.make_async_copy(rn_hbm.at[pl.ds(0, 2 * _C), :], bnr, sem).wait()

        def phase_a_group(g, carry):
            def phase_a_row(k, vecs):
                r = g * 16 + k
                m = iota == k
                out = []
                for (bh, bt, brn, s0) in ((bph, bpt, bpr, 0),
                                          (bnh, bnt, bnr, 6)):
                    h = _load4(bh, r)
                    t = _load4(bt, r)
                    rr_ = _load4(brn, 2 * r)
                    u = _load4(brn, 2 * r + 1)
                    for i, s in enumerate((_dot4(u, u), _dot4(h, h),
                                           _dot4(t, t), _dot4(rr_, rr_),
                                           _dot4(h, u), _dot4(t, u))):
                        out.append(jnp.where(m, jnp.full((16,), s),
                                             vecs[s0 + i]))
                return tuple(out[:6]) + tuple(out[6:])
            vecs = lax.fori_loop(0, 16, phase_a_row, (zeros,) * 12)
            for i in range(12):
                dots[i, pl.ds(g * 16, 16)] = vecs[i]
            return carry

        lax.fori_loop(0, _C // 16, phase_a_group, 0)

        def phase_b(g, carry):
            sl = pl.ds(g * 16, 16)
            for s0, c0 in ((0, 0), (6, 4)):
                nn = dots[s0 + 0, sl]
                hh = dots[s0 + 1, sl]
                tt = dots[s0 + 2, sl]
                rr_ = dots[s0 + 3, sl]
                hn = dots[s0 + 4, sl]
                tn = dots[s0 + 5, sl]
                inv_n = _rsqrt16(jnp.maximum(nn, _EPS))
                sq = nn * inv_n * inv_n
                a_h = hn * inv_n
                a_t = tn * inv_n
                php = hh - a_h * a_h * (2.0 - sq)
                ptp = tt - a_t * a_t * (2.0 - sq)
                inv_h = _rsqrt16(jnp.maximum(php, _EPS))
                inv_t = _rsqrt16(jnp.maximum(ptp, _EPS))
                inv_r = _rsqrt16(jnp.maximum(rr_, _EPS))
                gamma = inv_h * a_h * inv_n - inv_t * a_t * inv_n
                coef[c0 + 0, sl] = inv_h
                coef[c0 + 1, sl] = inv_t
                coef[c0 + 2, sl] = inv_r
                coef[c0 + 3, sl] = gamma
            return carry

        lax.fori_loop(0, _C // 16, phase_b, 0)

        def phase_c_row(r, a):
            g = r // 16
            k = r - g * 16
            m = iota == k
            mf = jnp.where(m, 1.0, 0.0)
            sl = pl.ds(g * 16, 16)
            sc = []
            for (bh, bt, brn, c0) in ((bph, bpt, bpr, 0),
                                      (bnh, bnt, bnr, 4)):
                inv_h = jnp.sum(coef[c0 + 0, sl] * mf)
                inv_t = jnp.sum(coef[c0 + 1, sl] * mf)
                inv_r = jnp.sum(coef[c0 + 2, sl] * mf)
                gamma = jnp.sum(coef[c0 + 3, sl] * mf)
                h = _load4(bh, r)
                t = _load4(bt, r)
                rr_ = _load4(brn, 2 * r)
                u = _load4(brn, 2 * r + 1)
                c = [jnp.abs(inv_h * h[j] - inv_t * t[j]
                             + inv_r * rr_[j] - gamma * u[j])
                     for j in range(4)]
                sc.append(jnp.sum(c[0] + c[1] + c[2] + c[3]))
            contrib = jnp.maximum(sc[0] - sc[1] + _MARGIN, 0.0)
            return a + jnp.where(m, jnp.full((16,), contrib), zeros)

        return lax.fori_loop(0, _C, phase_c_row, acc)

    acc = lax.fori_loop(0, _NCHUNK, chunk_body, zeros)
    out_v[...] = acc
    pltpu.sync_copy(out_v, out_hbm.at[wid])


@jax.jit
def _transh_sc(pos_h, pos_t, pos_r, neg_h, neg_t, neg_r,
               ent_emb, rel_emb, norm_vec):
    f32 = jnp.float32
    i32 = jnp.int32
    rn = jnp.stack([rel_emb, norm_vec], axis=1).reshape(-1, _DIM)
    call = pl.kernel(
        _transh_body,
        out_type=jax.ShapeDtypeStruct((_NW, 16), f32),
        mesh=plsc.VectorSubcoreMesh(core_axis_name="c", subcore_axis_name="s"),
        compiler_params=pltpu.CompilerParams(
            needs_layout_passes=False, use_tc_tiling_on_sc=True),
        scratch_types=[
            pltpu.VMEM((_C,), i32),
            pltpu.VMEM((_C,), i32),
            pltpu.VMEM((_C,), i32),
            pltpu.VMEM((_C,), i32),
            pltpu.VMEM((_C,), i32),
            pltpu.VMEM((_C,), i32),
            pltpu.VMEM((_C, _DIM), f32),
            pltpu.VMEM((_C, _DIM), f32),
            pltpu.VMEM((_C, _DIM), f32),
            pltpu.VMEM((_C, _DIM), f32),
            pltpu.VMEM((2 * _C, _DIM), f32),
            pltpu.VMEM((2 * _C, _DIM), f32),
            pltpu.VMEM((12, _C), f32),
            pltpu.VMEM((8, _C), f32),
            pltpu.VMEM((2, _C), f32),
            pltpu.VMEM((16,), f32),
            pltpu.SemaphoreType.DMA,
        ],
    )
    return call(pos_h, pos_t, neg_h, neg_t, pos_r, neg_r, ent_emb, rn)


def kernel(pos_h, pos_t, pos_r, neg_h, neg_t, neg_r,
           ent_emb, rel_emb, norm_vec):
    partials = _transh_sc(pos_h, pos_t, pos_r, neg_h, neg_t, neg_r,
                          ent_emb, rel_emb, norm_vec)
    return jnp.sum(partials) / _B
